# baseline (speedup 1.0000x reference)
import functools

import jax
import jax.numpy as jnp
from jax import lax
from jax.experimental import pallas as pl
from jax.experimental.pallas import tpu as pltpu
from jax.experimental.pallas import tpu_sc as plsc

_CHUNK = 128


@functools.cache
def _build(B, D):
    info = plsc.get_sparse_core_info()
    NC, NS = info.num_cores, info.num_subcores
    NW = NC * NS
    b_per_w = B // NW
    n_chunks = b_per_w // _CHUNK

    mesh = plsc.VectorSubcoreMesh(core_axis_name="c", subcore_axis_name="s")

    def body(mem_hbm, lu_hbm, idx_hbm, out_mem, out_lu,
             idx_v, rows_v, lu_v, gsems, lsem, wsem, isem):
        wid = lax.axis_index("s") * NC + lax.axis_index("c")
        base = wid * b_per_w
        pltpu.sync_copy(idx_hbm.at[pl.ds(wid * n_chunks, n_chunks)], idx_v)
        gathers = [pltpu.async_copy(
            mem_hbm.at[idx_v.at[j]],
            rows_v.at[pl.ds(j * _CHUNK, _CHUNK)], gsems.at[j])
            for j in range(n_chunks)]
        for c in gathers:
            c.wait()
        w = pltpu.async_copy(rows_v, out_mem.at[pl.ds(base, b_per_w)], wsem)
        lu_copies = [pltpu.async_copy(
            lu_hbm.at[idx_v.at[j]],
            lu_v.at[pl.ds(j * _CHUNK, _CHUNK)], lsem)
            for j in range(n_chunks)]
        for c in lu_copies:
            c.wait()
        pltpu.sync_copy(lu_v, out_lu.at[pl.ds(base, b_per_w)])
        w.wait()

    return pl.kernel(
        body,
        out_type=(jax.ShapeDtypeStruct((B, D), jnp.float32),
                  jax.ShapeDtypeStruct((B,), jnp.int32)),
        mesh=mesh,
        scratch_types=[
            pltpu.VMEM((n_chunks, _CHUNK), jnp.int32),
            pltpu.VMEM((b_per_w, D), jnp.float32),
            pltpu.VMEM((b_per_w,), jnp.int32),
            pltpu.SemaphoreType.DMA((n_chunks,)),
            pltpu.SemaphoreType.DMA,
            pltpu.SemaphoreType.DMA,
            pltpu.SemaphoreType.DMA,
        ],
    )


def kernel(memory, last_update, n_id):
    B = n_id.shape[0]
    D = memory.shape[1]
    k = _build(B, D)
    idx2d = n_id.astype(jnp.int32).reshape(B // _CHUNK, _CHUNK)
    mem_out, lu_out = k(memory, last_update.astype(jnp.int32), idx2d)
    return (mem_out, lu_out)


# speedup vs baseline: 1.0117x; 1.0117x over previous
import functools

import jax
import jax.numpy as jnp
from jax import lax
from jax.experimental import pallas as pl
from jax.experimental.pallas import tpu as pltpu
from jax.experimental.pallas import tpu_sc as plsc

_CHUNK = 128


@functools.cache
def _build(B, D):
    info = plsc.get_sparse_core_info()
    NC, NS = info.num_cores, info.num_subcores
    NW = NC * NS
    b_per_w = B // NW
    n_chunks = b_per_w // _CHUNK

    mesh = plsc.VectorSubcoreMesh(core_axis_name="c", subcore_axis_name="s")

    def body(mem_hbm, lu_hbm, idx_hbm, out_mem, out_lu,
             idx_v, rows_v, lu_v, gsems, lsem, wsem):
        wid = lax.axis_index("s") * NC + lax.axis_index("c")
        base = wid * b_per_w
        pltpu.sync_copy(idx_hbm.at[pl.ds(wid * n_chunks, n_chunks)], idx_v)
        gathers = [pltpu.async_copy(
            mem_hbm.at[idx_v.at[j]],
            rows_v.at[pl.ds(j * _CHUNK, _CHUNK)], gsems.at[j])
            for j in range(n_chunks)]
        lu_copies = [pltpu.async_copy(
            lu_hbm.at[idx_v.at[j]],
            lu_v.at[pl.ds(j * _CHUNK, _CHUNK)], lsem)
            for j in range(n_chunks)]
        for c in gathers:
            c.wait()
        w = pltpu.async_copy(rows_v, out_mem.at[pl.ds(base, b_per_w)], wsem)
        for c in lu_copies:
            c.wait()
        pltpu.sync_copy(lu_v, out_lu.at[pl.ds(base, b_per_w)])
        w.wait()

    return pl.kernel(
        body,
        out_type=(jax.ShapeDtypeStruct((B, D), jnp.float32),
                  jax.ShapeDtypeStruct((B,), jnp.int32)),
        mesh=mesh,
        scratch_types=[
            pltpu.VMEM((n_chunks, _CHUNK), jnp.int32),
            pltpu.VMEM((b_per_w, D), jnp.float32),
            pltpu.VMEM((b_per_w,), jnp.int32),
            pltpu.SemaphoreType.DMA((n_chunks,)),
            pltpu.SemaphoreType.DMA,
            pltpu.SemaphoreType.DMA,
        ],
    )


def kernel(memory, last_update, n_id):
    B = n_id.shape[0]
    D = memory.shape[1]
    k = _build(B, D)
    idx2d = n_id.astype(jnp.int32).reshape(B // _CHUNK, _CHUNK)
    mem_out, lu_out = k(memory, last_update.astype(jnp.int32), idx2d)
    return (mem_out, lu_out)
